# baseline (speedup 1.0000x reference)
import functools

import jax
import jax.numpy as jnp
from jax import lax
from jax.experimental import pallas as pl
from jax.experimental.pallas import tpu as pltpu
from jax.experimental.pallas import tpu_sc as plsc


def _make_sc_kernel(B, S, D):
    info = plsc.get_sparse_core_info()
    NC, NS = info.num_cores, info.num_subcores
    NW = NC * NS
    rows_per_w = S // NW
    CH = 8
    n_ch = rows_per_w // CH
    mesh = plsc.VectorSubcoreMesh(core_axis_name="c", subcore_axis_name="s")

    NBUF = 6

    @functools.partial(
        pl.kernel,
        mesh=mesh,
        out_type=jax.ShapeDtypeStruct((B, S, D), jnp.float32),
        scratch_types=[pltpu.VMEM((CH, D), jnp.float32)] * NBUF
        + [pltpu.SemaphoreType.DMA] * (1 + NBUF),
    )
    def k(enc, out, *scratch):
        bufs = scratch[:NBUF]
        gsem = scratch[NBUF]
        ssems = scratch[NBUF + 1:]
        wid = lax.axis_index("s") * NC + lax.axis_index("c")
        base = wid * rows_per_w
        gathers = [None] * n_ch
        scatters = [None] * n_ch

        def start_gather(c):
            row = base + c * CH
            gathers[c] = pltpu.async_copy(
                enc.at[pl.ds(row, CH)], bufs[c % NBUF], gsem
            )

        def start_scatters(c):
            row = base + c * CH
            scatters[c] = [
                pltpu.async_copy(
                    bufs[c % NBUF], out.at[b, pl.ds(row, CH)], ssems[c % NBUF]
                )
                for b in range(B)
            ]

        for c in range(min(NBUF, n_ch)):
            start_gather(c)
        for c in range(n_ch):
            if c >= NBUF:
                for cp in scatters[c - NBUF]:
                    cp.wait()
                start_gather(c)
            gathers[c].wait()
            start_scatters(c)
        for c in range(max(0, n_ch - NBUF), n_ch):
            for cp in scatters[c]:
                cp.wait()

    return k


def kernel(x, encoding):
    B, S, D = x.shape
    return _make_sc_kernel(B, S, D)(encoding[:S])


# speedup vs baseline: 1.0400x; 1.0400x over previous
import functools

import jax
import jax.numpy as jnp
from jax import lax
from jax.experimental import pallas as pl
from jax.experimental.pallas import tpu as pltpu
from jax.experimental.pallas import tpu_sc as plsc


def _make_sc_kernel(B, S, D):
    info = plsc.get_sparse_core_info()
    NC, NS = info.num_cores, info.num_subcores
    NW = NC * NS
    rows_per_w = S // NW
    CH = 16
    n_ch = rows_per_w // CH
    mesh = plsc.VectorSubcoreMesh(core_axis_name="c", subcore_axis_name="s")

    NBUF = 3

    @functools.partial(
        pl.kernel,
        mesh=mesh,
        out_type=jax.ShapeDtypeStruct((B, S, D), jnp.float32),
        scratch_types=[pltpu.VMEM((CH, D), jnp.float32)] * NBUF
        + [pltpu.SemaphoreType.DMA] * (1 + NBUF),
    )
    def k(enc, out, *scratch):
        bufs = scratch[:NBUF]
        gsem = scratch[NBUF]
        ssems = scratch[NBUF + 1:]
        wid = lax.axis_index("s") * NC + lax.axis_index("c")
        base = wid * rows_per_w
        gathers = [None] * n_ch
        scatters = [None] * n_ch

        def start_gather(c):
            row = base + c * CH
            gathers[c] = pltpu.async_copy(
                enc.at[pl.ds(row, CH)], bufs[c % NBUF], gsem
            )

        def start_scatters(c):
            row = base + c * CH
            scatters[c] = [
                pltpu.async_copy(
                    bufs[c % NBUF], out.at[b, pl.ds(row, CH)], ssems[c % NBUF]
                )
                for b in range(B)
            ]

        for c in range(min(NBUF, n_ch)):
            start_gather(c)
        for c in range(n_ch):
            if c >= NBUF:
                for cp in scatters[c - NBUF]:
                    cp.wait()
                start_gather(c)
            gathers[c].wait()
            start_scatters(c)
        for c in range(max(0, n_ch - NBUF), n_ch):
            for cp in scatters[c]:
                cp.wait()

    return k


def kernel(x, encoding):
    B, S, D = x.shape
    return _make_sc_kernel(B, S, D)(encoding[:S])
